# baseline (speedup 1.0000x reference)
import functools

import jax
import jax.numpy as jnp
from jax import lax
from jax.experimental import pallas as pl
from jax.experimental.pallas import tpu as pltpu
from jax.experimental.pallas import tpu_sc as plsc

_NUM_ITEM = 1000000
_DIM = 64
_BATCH = 16384

_INFO = plsc.get_sparse_core_info()
_NC, _NS, _L = _INFO.num_cores, _INFO.num_subcores, _INFO.num_lanes
_NW = _NC * _NS
_BPW = _BATCH // _NW
_CHUNK = 32
_NCHUNKS = _BPW // _CHUNK


def _remap(v):
    v = jnp.where(v == -1, _NUM_ITEM, v)
    return jnp.minimum(v, _NUM_ITEM - 1)


def _body(
    idx_hbm, tbl_hbm, pad_hbm, out_hbm, idx_v, pad_v, val_v, rows_f, sem, semg
):
    wid = lax.axis_index("s") * _NC + lax.axis_index("c")
    base = wid * _BPW

    pltpu.sync_copy(idx_hbm.at[pl.ds(base, _BPW)], idx_v)
    pltpu.sync_copy(pad_hbm, pad_v)

    def fetch_chunk(c, buf):
        def fetch_block(j, carry2):
            vj = _remap(idx_v[pl.ds(c * _CHUNK + j * _L, _L)]) & ~7
            for k in range(_L):
                m = j * _L + k
                g8 = pl.multiple_of(vj[k], 8)
                pltpu.async_copy(
                    tbl_hbm.at[pl.ds(g8, 8), :],
                    val_v.at[pl.ds((buf * _CHUNK + m) * 8, 8), :],
                    semg,
                )
            return carry2

        lax.fori_loop(0, _CHUNK // _L, fetch_block, 0, unroll=False)

    def drain_chunk():
        pltpu.make_async_copy(
            tbl_hbm.at[pl.ds(0, _CHUNK * 8), :],
            val_v.at[pl.ds(0, _CHUNK * 8), :],
            semg,
        ).wait()

    def extract_chunk(c, buf):
        def extract_block(j, carry2):
            raw = idx_v[pl.ds(c * _CHUNK + j * _L, _L)]
            vj = _remap(raw) & 7
            for k in range(_L):
                m = j * _L + k
                row = (buf * _CHUNK + m) * 8 + vj[k]
                is_pad = raw[k] == -1
                dst = pl.multiple_of((c * _CHUNK + m) * _DIM, _DIM)
                for t in range(_DIM // _L):
                    r16 = val_v[row, pl.ds(t * _L, _L)]
                    p16 = pad_v[0, pl.ds(t * _L, _L)]
                    rows_f[pl.ds(dst + t * _L, _L)] = jnp.where(
                        is_pad, p16, r16
                    )
            return carry2

        lax.fori_loop(0, _CHUNK // _L, extract_block, 0, unroll=False)

    fetch_chunk(0, 0)

    def chunk_step(c, carry):
        drain_chunk()
        fetch_chunk(c + 1, (c + 1) % 2)
        extract_chunk(c, c % 2)
        return carry

    lax.fori_loop(0, _NCHUNKS - 1, chunk_step, 0, unroll=False)
    drain_chunk()
    extract_chunk(_NCHUNKS - 1, (_NCHUNKS - 1) % 2)

    pltpu.sync_copy(rows_f, out_hbm.at[pl.ds(base * _DIM, _BPW * _DIM)])


@jax.jit
def _gather_rows(items, table, pad_row):
    mesh = plsc.VectorSubcoreMesh(core_axis_name="c", subcore_axis_name="s")
    return pl.kernel(
        _body,
        mesh=mesh,
        compiler_params=pltpu.CompilerParams(use_tc_tiling_on_sc=True),
        out_type=jax.ShapeDtypeStruct((_BATCH * _DIM,), jnp.float32),
        scratch_types=[
            pltpu.VMEM((_BPW,), jnp.int32),
            pltpu.VMEM((8, _DIM), jnp.float32),
            pltpu.VMEM((2 * _CHUNK * 8, _DIM), jnp.float32),
            pltpu.VMEM((_BPW * _DIM,), jnp.float32),
            pltpu.SemaphoreType.DMA,
            pltpu.SemaphoreType.DMA,
        ],
    )(items, table, pad_row)


def kernel(obs, item_table):
    items = obs[:, 1].astype(jnp.int32)
    pads = jnp.take(
        item_table, jnp.full((_BATCH,), _NUM_ITEM, jnp.int32), axis=0
    )
    pad_row = pads[:8]
    out_flat = _gather_rows(items, item_table, pad_row)
    return out_flat.reshape(_BATCH, _DIM)


# speedup vs baseline: 2.5547x; 2.5547x over previous
import functools

import jax
import jax.numpy as jnp
from jax import lax
from jax.experimental import pallas as pl
from jax.experimental.pallas import tpu as pltpu
from jax.experimental.pallas import tpu_sc as plsc

_NUM_ITEM = 1000000
_DIM = 64
_BATCH = 16384

_INFO = plsc.get_sparse_core_info()
_NC, _NS, _L = _INFO.num_cores, _INFO.num_subcores, _INFO.num_lanes
_NW = _NC * _NS
_BPW = _BATCH // _NW
_CHUNK = 32
_NCHUNKS = _BPW // _CHUNK


def _remap(v):
    v = jnp.where(v == -1, _NUM_ITEM, v)
    return jnp.minimum(v, _NUM_ITEM - 1)


def _body(
    idx_hbm, tbl_hbm, pad_hbm, out_hbm, idx_v, pad_v, val_v, rows_f, sem, semg
):
    wid = lax.axis_index("s") * _NC + lax.axis_index("c")
    base = wid * _BPW

    pltpu.sync_copy(idx_hbm.at[pl.ds(base, _BPW)], idx_v)
    pltpu.sync_copy(pad_hbm, pad_v)

    def fetch_chunk(c, buf):
        def fetch_block(j, carry2):
            vj = _remap(idx_v[pl.ds(c * _CHUNK + j * _L, _L)]) & ~7
            for k in range(_L):
                m = j * _L + k
                g8 = pl.multiple_of(vj[k], 8)
                pltpu.async_copy(
                    tbl_hbm.at[pl.ds(g8, 8), :],
                    val_v.at[pl.ds((buf * _CHUNK + m) * 8, 8), :],
                    semg,
                )
            return carry2

        lax.fori_loop(0, _CHUNK // _L, fetch_block, 0, unroll=False)

    def drain_chunk():
        pltpu.make_async_copy(
            tbl_hbm.at[pl.ds(0, _CHUNK * 8), :],
            val_v.at[pl.ds(0, _CHUNK * 8), :],
            semg,
        ).wait()

    def extract_chunk(c, buf):
        def extract_block(j, carry2):
            raw = idx_v[pl.ds(c * _CHUNK + j * _L, _L)]
            vj = _remap(raw) & 7
            for k in range(_L):
                m = j * _L + k
                row = (buf * _CHUNK + m) * 8 + vj[k]
                is_pad = raw[k] == -1
                dst = pl.multiple_of((c * _CHUNK + m) * _DIM, _DIM)
                for t in range(_DIM // _L):
                    r16 = val_v[row, pl.ds(t * _L, _L)]
                    p16 = pad_v[0, pl.ds(t * _L, _L)]
                    rows_f[pl.ds(dst + t * _L, _L)] = jnp.where(
                        is_pad, p16, r16
                    )
            return carry2

        lax.fori_loop(0, _CHUNK // _L, extract_block, 0, unroll=False)

    fetch_chunk(0, 0)

    def chunk_step(c, carry):
        drain_chunk()
        fetch_chunk(c + 1, (c + 1) % 2)
        extract_chunk(c, c % 2)
        return carry

    lax.fori_loop(0, _NCHUNKS - 1, chunk_step, 0, unroll=False)
    drain_chunk()
    extract_chunk(_NCHUNKS - 1, (_NCHUNKS - 1) % 2)

    pltpu.sync_copy(rows_f, out_hbm.at[pl.ds(base * _DIM, _BPW * _DIM)])


@jax.jit
def _gather_rows(items, table, pad_row):
    mesh = plsc.VectorSubcoreMesh(core_axis_name="c", subcore_axis_name="s")
    return pl.kernel(
        _body,
        mesh=mesh,
        compiler_params=pltpu.CompilerParams(use_tc_tiling_on_sc=True),
        out_type=jax.ShapeDtypeStruct((_BATCH * _DIM,), jnp.float32),
        scratch_types=[
            pltpu.VMEM((_BPW,), jnp.int32),
            pltpu.VMEM((8, _DIM), jnp.float32),
            pltpu.VMEM((2 * _CHUNK * 8, _DIM), jnp.float32),
            pltpu.VMEM((_BPW * _DIM,), jnp.float32),
            pltpu.SemaphoreType.DMA,
            pltpu.SemaphoreType.DMA,
        ],
    )(items, table, pad_row)


def kernel(obs, item_table):
    items = obs[:, 1].astype(jnp.int32)
    pad_row = jnp.take(
        item_table, jnp.full((8,), _NUM_ITEM, jnp.int32), axis=0
    )
    out_flat = _gather_rows(items, item_table, pad_row)
    return out_flat.reshape(_BATCH, _DIM)
